# baseline (speedup 1.0000x reference)
import functools

import jax
import jax.numpy as jnp
from jax import lax
from jax.experimental import pallas as pl
from jax.experimental.pallas import tpu as pltpu
from jax.experimental.pallas import tpu_sc as plsc

N = 10000
E = 320000
D_IN = 128
D_HID = 128
D_OUT = 64

NC = 2
NS = 16
CH = 128
CPT = 168
EP = NS * CPT * CH
SPLIT = 6000
NACC = 6144
TRASH = 6100
STRIPE = NACC // NS
RB = 400
SB = SPLIT // RB


def _sc_mesh():
    return plsc.VectorSubcoreMesh(
        core_axis_name="c", subcore_axis_name="s", num_cores=NC, num_subcores=NS
    )


def _zero_acc(acc, zeros, zb, s):
    pltpu.sync_copy(zeros, zb)
    for k in range(NACC // NS // 128):
        pltpu.sync_copy(zb, acc.at[pl.ds((s * (NACC // NS // 128) + k) * 128, 128)])


def _copy_out(acc, out, c, s):
    pltpu.sync_copy(
        acc.at[pl.ds(s * STRIPE, STRIPE)], out.at[c].at[pl.ds(s * STRIPE, STRIPE)]
    )


def _remap_dst(didx, c, s):
    base = c * SPLIT
    trash = (SPLIT - c * 1700) + s
    lim = (SPLIT + c * (NACC - SPLIT)).astype(jnp.uint32)

    @pl.loop(0, CPT)
    def _(g):
        for k in range(CH // 16):
            t = didx[g, pl.ds(k * 16, 16)] - base
            keep = t.astype(jnp.uint32) < lim
            didx[g, pl.ds(k * 16, 16)] = jnp.where(keep, t, trash)


def _split_map(i):
    return jnp.where(i < SB, 0, 1), jnp.where(i < SB, i, i - SB)


@functools.lru_cache(maxsize=None)
def _make_deg_kernel():
    @functools.partial(
        pl.kernel,
        out_type=jax.ShapeDtypeStruct((NC, NACC, 16), jnp.float32),
        mesh=_sc_mesh(),
        scratch_types=[
            pltpu.VMEM((CPT, CH), jnp.int32),
            pltpu.VMEM((CH, 16), jnp.float32),
            pltpu.VMEM((128, 16), jnp.float32),
            pltpu.VMEM_SHARED((NACC, 16), jnp.float32),
            pltpu.SemaphoreType.DMA,
            pltpu.SemaphoreType.DMA,
            pltpu.SemaphoreType.DMA,
            pltpu.SemaphoreType.DMA,
        ],
    )
    def deg_kernel(dst3, ones, zeros, out, didx, ones_v, zb, acc, s0, s1, s2, s3):
        c = lax.axis_index("c")
        s = lax.axis_index("s")
        pltpu.sync_copy(dst3.at[s], didx)
        _remap_dst(didx, c, s)
        pltpu.sync_copy(ones, ones_v)
        _zero_acc(acc, zeros, zb, s)
        plsc.subcore_barrier()

        sems = (s0, s1, s2, s3)
        for j in range(4):
            pltpu.async_copy(ones_v, acc.at[didx.at[j]], sems[j], add=True)

        @pl.loop(4, CPT, step=4)
        def _(g):
            for j in range(4):
                pltpu.make_async_copy(ones_v, acc.at[didx.at[g - 4 + j]], sems[j]).wait()
                pltpu.async_copy(ones_v, acc.at[didx.at[g + j]], sems[j], add=True)

        for j in range(4):
            pltpu.make_async_copy(ones_v, acc.at[didx.at[CPT - 4 + j]], sems[j]).wait()

        plsc.subcore_barrier()
        _copy_out(acc, out, c, s)

    return deg_kernel


@functools.lru_cache(maxsize=None)
def _make_agg_kernel():
    W = 128

    @functools.partial(
        pl.kernel,
        out_type=jax.ShapeDtypeStruct((NC, NACC, W), jnp.float32),
        mesh=_sc_mesh(),
        scratch_types=[
            pltpu.VMEM((2, 32, CH), jnp.int32),
            pltpu.VMEM((CPT, CH), jnp.int32),
            pltpu.VMEM((3, CH, W), jnp.float32),
            pltpu.VMEM_SHARED((NACC, W), jnp.float32),
            pltpu.SemaphoreType.DMA,
            pltpu.SemaphoreType.DMA,
            pltpu.SemaphoreType.DMA,
            pltpu.SemaphoreType.DMA,
            pltpu.SemaphoreType.DMA,
            pltpu.SemaphoreType.DMA,
        ],
    )
    def agg_kernel(
        table, src3, dst3, out, swin, didx, rows, acc,
        g0, g1, g2, q0, q1, q2,
    ):
        c = lax.axis_index("c")
        s = lax.axis_index("s")
        gsems = (g0, g1, g2)
        ssems = (q0, q1, q2)
        pltpu.sync_copy(src3.at[s, pl.ds(0, 32)], swin.at[0])
        pltpu.sync_copy(dst3.at[s], didx)
        _remap_dst(didx, c, s)

        def gather(ch, b):
            pltpu.async_copy(
                table.at[swin.at[(ch // 32) % 2, ch % 32]], rows.at[b], gsems[b]
            )

        def wait_gather(ch, b):
            pltpu.make_async_copy(
                table.at[swin.at[(ch // 32) % 2, ch % 32]], rows.at[b], gsems[b]
            ).wait()

        @pl.loop(0, CH)
        def _(r):
            for k in range(W // 16):
                rows[0, r, pl.ds(k * 16, 16)] = jnp.zeros((16,), jnp.float32)

        for k in range(NACC // NS // 128):
            pltpu.sync_copy(
                rows.at[0], acc.at[pl.ds((s * (NACC // NS // 128) + k) * 128, 128)]
            )
        plsc.subcore_barrier()

        for b in range(3):
            gather(b, b)

        def triple(g, last):
            descs = []
            for j in range(3):
                ch = g + j
                wait_gather(ch, j)
                descs.append(
                    pltpu.async_copy(rows.at[j], acc.at[didx.at[ch]], ssems[j], add=True)
                )
            if not last:
                w = (g + 5) // 32

                @pl.when(jnp.logical_and(w * 32 >= g + 3, w * 32 < CPT))
                def _():
                    pltpu.sync_copy(src3.at[s, pl.ds(w * 32, 32)], swin.at[w % 2])

            for j in range(3):
                descs[j].wait()
                if not last:
                    gather(g + 3 + j, j)

        @pl.loop(0, CPT - 3, step=3)
        def _(g):
            triple(g, last=False)

        triple(CPT - 3, last=True)

        plsc.subcore_barrier()
        _copy_out(acc, out, c, s)

    return agg_kernel


def _norm_from_deg(dblk):
    return lax.rsqrt(jnp.maximum(dblk[0, :, 0:1], 1.0))


def _dense1_body(x_ref, w_ref, d_ref, out_ref):
    norm = _norm_from_deg(d_ref[...])
    out_ref[...] = (
        jnp.dot(x_ref[...], w_ref[...], preferred_element_type=jnp.float32) * norm
    )


def _dense2_body(a_ref, d_ref, b1_ref, w_ref, out_ref):
    norm = _norm_from_deg(d_ref[...])
    h1 = jax.nn.relu(a_ref[0] * norm + b1_ref[...])
    h2 = jnp.dot(h1, w_ref[...], preferred_element_type=jnp.float32) * norm
    out_ref[:, 0:64] = h2
    out_ref[:, 64:128] = jnp.zeros_like(h2)


def _dense3_body(a_ref, d_ref, b2_ref, out_ref):
    norm = _norm_from_deg(d_ref[...])
    out_ref[...] = jax.nn.relu(a_ref[0, :, 0:64] * norm + b2_ref[...])


def _dense1(x, w1, deg):
    return pl.pallas_call(
        _dense1_body,
        grid=(N // RB,),
        in_specs=[
            pl.BlockSpec((RB, D_IN), lambda i: (i, 0)),
            pl.BlockSpec((D_IN, D_HID), lambda i: (0, 0)),
            pl.BlockSpec((1, RB, 16), lambda i: (*_split_map(i), 0)),
        ],
        out_specs=pl.BlockSpec((RB, D_HID), lambda i: (i, 0)),
        out_shape=jax.ShapeDtypeStruct((N, D_HID), jnp.float32),
    )(x, w1, deg)


def _dense2(agg1, deg, b1, w2):
    return pl.pallas_call(
        _dense2_body,
        grid=(N // RB,),
        in_specs=[
            pl.BlockSpec((1, RB, 128), lambda i: (*_split_map(i), 0)),
            pl.BlockSpec((1, RB, 16), lambda i: (*_split_map(i), 0)),
            pl.BlockSpec((1, D_HID), lambda i: (0, 0)),
            pl.BlockSpec((D_HID, D_OUT), lambda i: (0, 0)),
        ],
        out_specs=pl.BlockSpec((RB, 128), lambda i: (i, 0)),
        out_shape=jax.ShapeDtypeStruct((N, 128), jnp.float32),
    )(agg1, deg, b1, w2)


def _dense3(agg2, deg, b2):
    return pl.pallas_call(
        _dense3_body,
        grid=(N // RB,),
        in_specs=[
            pl.BlockSpec((1, RB, 128), lambda i: (*_split_map(i), 0)),
            pl.BlockSpec((1, RB, 16), lambda i: (*_split_map(i), 0)),
            pl.BlockSpec((1, D_OUT), lambda i: (0, 0)),
        ],
        out_specs=pl.BlockSpec((RB, D_OUT), lambda i: (0, 0)),
        out_shape=jax.ShapeDtypeStruct((N, D_OUT), jnp.float32),
    )(agg2, deg, b2)


def kernel(features, edge_index, W1, b1, W2, b2):
    src = edge_index[0]
    dst = edge_index[1]
    pad = EP - E
    srcp = jnp.concatenate([src, jnp.zeros((pad,), jnp.int32)])
    dstp = jnp.concatenate([dst, jnp.full((pad,), N, jnp.int32)])
    src3 = srcp.reshape(NS, CPT, CH)
    dst3 = dstp.reshape(NS, CPT, CH)
    ones16 = jnp.ones((CH, 16), jnp.float32)
    z16 = jnp.zeros((128, 16), jnp.float32)

    deg = _make_deg_kernel()(dst3, ones16, z16)
    hs1 = _dense1(features, W1, deg)
    agg1 = _make_agg_kernel()(hs1, src3, dst3)
    hs2 = _dense2(agg1, deg, b1.reshape(1, D_HID), W2)
    agg2 = _make_agg_kernel()(hs2, src3, dst3)
    return _dense3(agg2, deg, b2.reshape(1, D_OUT))


# speedup vs baseline: 2.4995x; 2.4995x over previous
import functools

import jax
import jax.numpy as jnp
from jax import lax
from jax.experimental import pallas as pl
from jax.experimental.pallas import tpu as pltpu
from jax.experimental.pallas import tpu_sc as plsc

N = 10000
E = 320000
D_IN = 128
D_HID = 128
D_OUT = 64

NC = 2
NS = 16
CH = 128
CPT = 160
EP = NS * CPT * CH
SPLIT = 6000
NACC = 6144
TRASH = 6100
STRIPE = NACC // NS
RB = 400
SB = SPLIT // RB


def _sc_mesh():
    return plsc.VectorSubcoreMesh(
        core_axis_name="c", subcore_axis_name="s", num_cores=NC, num_subcores=NS
    )


def _zero_acc(acc, zeros, zb, s):
    pltpu.sync_copy(zeros, zb)
    for k in range(NACC // NS // 128):
        pltpu.sync_copy(zb, acc.at[pl.ds((s * (NACC // NS // 128) + k) * 128, 128)])


def _copy_out(acc, out, c, s):
    pltpu.sync_copy(
        acc.at[pl.ds(s * STRIPE, STRIPE)], out.at[c].at[pl.ds(s * STRIPE, STRIPE)]
    )


def _split_map(i):
    return jnp.where(i < SB, 0, 1), jnp.where(i < SB, i, i - SB)


@functools.lru_cache(maxsize=None)
def _make_deg_kernel():
    @functools.partial(
        pl.kernel,
        out_type=jax.ShapeDtypeStruct((NC, NACC, 16), jnp.float32),
        mesh=_sc_mesh(),
        scratch_types=[
            pltpu.VMEM((CPT, CH), jnp.int32),
            pltpu.VMEM((CH, 16), jnp.float32),
            pltpu.VMEM((128, 16), jnp.float32),
            pltpu.VMEM_SHARED((NACC, 16), jnp.float32),
            pltpu.SemaphoreType.DMA,
            pltpu.SemaphoreType.DMA,
            pltpu.SemaphoreType.DMA,
            pltpu.SemaphoreType.DMA,
        ],
    )
    def deg_kernel(dst4, ones, zeros, out, didx, ones_v, zb, acc, s0, s1, s2, s3):
        c = lax.axis_index("c")
        s = lax.axis_index("s")
        pltpu.sync_copy(dst4.at[c, s], didx)
        pltpu.sync_copy(ones, ones_v)
        _zero_acc(acc, zeros, zb, s)
        plsc.subcore_barrier()

        sems = (s0, s1, s2, s3)
        for j in range(4):
            pltpu.async_copy(ones_v, acc.at[didx.at[j]], sems[j], add=True)

        @pl.loop(4, CPT, step=4)
        def _(g):
            for j in range(4):
                pltpu.make_async_copy(ones_v, acc.at[didx.at[g - 4 + j]], sems[j]).wait()
                pltpu.async_copy(ones_v, acc.at[didx.at[g + j]], sems[j], add=True)

        for j in range(4):
            pltpu.make_async_copy(ones_v, acc.at[didx.at[CPT - 4 + j]], sems[j]).wait()

        plsc.subcore_barrier()
        _copy_out(acc, out, c, s)

    return deg_kernel


@functools.lru_cache(maxsize=None)
def _make_agg_kernel():
    W = 128
    SCPT = CPT // 2
    WIN = 8

    @functools.partial(
        pl.kernel,
        out_type=jax.ShapeDtypeStruct((NC, NACC, W), jnp.float32),
        mesh=_sc_mesh(),
        scratch_types=[
            pltpu.VMEM((2, WIN, 2, CH), jnp.int32),
            pltpu.VMEM((2, WIN, 2, CH), jnp.int32),
            pltpu.VMEM((2, 2 * CH, W), jnp.float32),
            pltpu.VMEM_SHARED((NACC, W), jnp.float32),
            pltpu.SemaphoreType.DMA,
            pltpu.SemaphoreType.DMA,
        ],
    )
    def agg_kernel(table, src5, dst5, out, swin, dwin, rows, acc, g0, g1):
        c = lax.axis_index("c")
        s = lax.axis_index("s")
        gsems = (g0, g1)

        def load_windows(w):
            pltpu.sync_copy(src5.at[s, pl.ds(w * WIN, WIN)], swin.at[w % 2])
            pltpu.sync_copy(dst5.at[c, s, pl.ds(w * WIN, WIN)], dwin.at[w % 2])

        def gather2(sc, b):
            for k in range(2):
                pltpu.async_copy(
                    table.at[swin.at[(sc // WIN) % 2, sc % WIN, k]],
                    rows.at[b, pl.ds(k * CH, CH)],
                    gsems[b],
                )

        def wait_gather2(sc, b):
            for k in range(2):
                pltpu.make_async_copy(
                    table.at[swin.at[(sc // WIN) % 2, sc % WIN, k]],
                    rows.at[b, pl.ds(k * CH, CH)],
                    gsems[b],
                ).wait()

        def scatter2(sc, b):
            for k in range(2):
                pltpu.sync_copy(
                    rows.at[b, pl.ds(k * CH, CH)],
                    acc.at[dwin.at[(sc // WIN) % 2, sc % WIN, k]],
                    add=True,
                )

        @pl.loop(0, 2 * CH)
        def _(r):
            for k in range(W // 16):
                rows[0, r, pl.ds(k * 16, 16)] = jnp.zeros((16,), jnp.float32)

        pltpu.sync_copy(rows.at[0], acc.at[pl.ds(s * STRIPE, 2 * CH)])
        pltpu.sync_copy(
            rows.at[0, pl.ds(0, STRIPE - 2 * CH)],
            acc.at[pl.ds(s * STRIPE + 2 * CH, STRIPE - 2 * CH)],
        )
        plsc.subcore_barrier()

        load_windows(0)
        gather2(0, 0)
        gather2(1, 1)

        @pl.loop(0, SCPT - 2, step=2)
        def _(g):
            for j in range(2):
                sc = g + j
                wait_gather2(sc, j)
                scatter2(sc, j)
                nxt = sc + 2

                @pl.when(jnp.logical_and(nxt % WIN == 0, nxt < SCPT))
                def _():
                    load_windows(nxt // WIN)

                gather2(nxt, j)

        for j in range(2):
            sc = SCPT - 2 + j
            wait_gather2(sc, j)
            scatter2(sc, j)

        plsc.subcore_barrier()
        _copy_out(acc, out, c, s)

    return agg_kernel


def _norm_from_deg(dblk):
    return lax.rsqrt(jnp.maximum(dblk[0, :, 0:1], 1.0))


def _dense1_body(x_ref, w_ref, d_ref, out_ref):
    norm = _norm_from_deg(d_ref[...])
    out_ref[...] = (
        jnp.dot(x_ref[...], w_ref[...], preferred_element_type=jnp.float32) * norm
    )


def _dense2_body(a_ref, d_ref, b1_ref, w_ref, out_ref):
    norm = _norm_from_deg(d_ref[...])
    h1 = jax.nn.relu(a_ref[0] * norm + b1_ref[...])
    h2 = jnp.dot(h1, w_ref[...], preferred_element_type=jnp.float32) * norm
    out_ref[:, 0:64] = h2
    out_ref[:, 64:128] = jnp.zeros_like(h2)


def _dense3_body(a_ref, d_ref, b2_ref, out_ref):
    norm = _norm_from_deg(d_ref[...])
    out_ref[...] = jax.nn.relu(a_ref[0, :, 0:64] * norm + b2_ref[...])


def _dense1(x, w1, deg):
    return pl.pallas_call(
        _dense1_body,
        grid=(N // RB,),
        in_specs=[
            pl.BlockSpec((RB, D_IN), lambda i: (i, 0)),
            pl.BlockSpec((D_IN, D_HID), lambda i: (0, 0)),
            pl.BlockSpec((1, RB, 16), lambda i: (*_split_map(i), 0)),
        ],
        out_specs=pl.BlockSpec((RB, D_HID), lambda i: (i, 0)),
        out_shape=jax.ShapeDtypeStruct((N, D_HID), jnp.float32),
    )(x, w1, deg)


def _dense2(agg1, deg, b1, w2):
    return pl.pallas_call(
        _dense2_body,
        grid=(N // RB,),
        in_specs=[
            pl.BlockSpec((1, RB, 128), lambda i: (*_split_map(i), 0)),
            pl.BlockSpec((1, RB, 16), lambda i: (*_split_map(i), 0)),
            pl.BlockSpec((1, D_HID), lambda i: (0, 0)),
            pl.BlockSpec((D_HID, D_OUT), lambda i: (0, 0)),
        ],
        out_specs=pl.BlockSpec((RB, 128), lambda i: (i, 0)),
        out_shape=jax.ShapeDtypeStruct((N, 128), jnp.float32),
    )(agg1, deg, b1, w2)


def _dense3(agg2, deg, b2):
    return pl.pallas_call(
        _dense3_body,
        grid=(N // RB,),
        in_specs=[
            pl.BlockSpec((1, RB, 128), lambda i: (*_split_map(i), 0)),
            pl.BlockSpec((1, RB, 16), lambda i: (*_split_map(i), 0)),
            pl.BlockSpec((1, D_OUT), lambda i: (0, 0)),
        ],
        out_specs=pl.BlockSpec((RB, D_OUT), lambda i: (0, 0)),
        out_shape=jax.ShapeDtypeStruct((N, D_OUT), jnp.float32),
    )(agg2, deg, b2)


def kernel(features, edge_index, W1, b1, W2, b2):
    src = edge_index[0]
    dst = edge_index[1]
    pad = EP - E
    srcp = jnp.concatenate([src, jnp.zeros((pad,), jnp.int32)])
    dstp = jnp.concatenate([dst, jnp.full((pad,), N, jnp.int32)])
    dst_r = dstp.reshape(NS, CPT * CH)
    trash0 = SPLIT + jnp.arange(NS, dtype=jnp.int32)[:, None]
    trash1 = (SPLIT - 1700) + jnp.arange(NS, dtype=jnp.int32)[:, None]
    d0 = jnp.where(dst_r < SPLIT, dst_r, trash0)
    d1 = jnp.where(dst_r >= SPLIT, dst_r - SPLIT, trash1)
    dst4 = jnp.stack([d0, d1]).reshape(NC, NS, CPT, CH)
    src5 = srcp.reshape(NS, CPT // 2, 2, CH)
    dst5 = dst4.reshape(NC, NS, CPT // 2, 2, CH)
    ones16 = jnp.ones((CH, 16), jnp.float32)
    z16 = jnp.zeros((128, 16), jnp.float32)

    deg = _make_deg_kernel()(dst4, ones16, z16)
    hs1 = _dense1(features, W1, deg)
    agg1 = _make_agg_kernel()(hs1, src5, dst5)
    hs2 = _dense2(agg1, deg, b1.reshape(1, D_HID), W2)
    agg2 = _make_agg_kernel()(hs2, src5, dst5)
    return _dense3(agg2, deg, b2.reshape(1, D_OUT))
